# Initial kernel scaffold; baseline (speedup 1.0000x reference)
import jax
import jax.numpy as jnp
from jax.experimental import pallas as pl


def kernel(hidden_states, w_gate, w_gate_proj, w_up_proj, w_down_proj):
    raise NotImplementedError("write your pallas kernel here")


# speedup vs baseline: 2.3390x; 2.3390x over previous
import functools
import math

import jax
import jax.numpy as jnp
from jax import lax
from jax.experimental import pallas as pl
from jax.experimental.pallas import tpu as pltpu

_K = 2


def _cumsum_excl(x, T):
    acc = x
    k = 1
    while k < T:
        shifted = jnp.concatenate(
            [jnp.zeros((k,) + x.shape[1:], x.dtype), acc[:-k]], axis=0)
        acc = acc + shifted
        k *= 2
    return acc - x


def _routing_body(cap, h_ref, wg_ref, ids_ref, wk_ref, rank_ref, hb_ref):
    h = h_ref[...]
    wg = wg_ref[...]
    T = h.shape[0]
    E = wg.shape[1]
    logits = jnp.dot(h.astype(jnp.bfloat16), wg.astype(jnp.bfloat16),
                     preferred_element_type=jnp.float32)
    lane = lax.broadcasted_iota(jnp.int32, (T, E), 1)

    m = jnp.max(logits, axis=-1, keepdims=True)
    p = jnp.exp(logits - m)
    s = jnp.sum(p, axis=-1, keepdims=True)
    probs = p / s

    m1 = jnp.max(probs, axis=-1, keepdims=True)
    i1 = jnp.min(jnp.where(probs == m1, lane, E), axis=-1, keepdims=True)
    masked = jnp.where(lane == i1, -jnp.inf, probs)
    m2 = jnp.max(masked, axis=-1, keepdims=True)
    i2 = jnp.min(jnp.where(masked == m2, lane, E), axis=-1, keepdims=True)
    denom = m1 + m2
    w1 = m1 / denom
    w2 = m2 / denom
    ids_ref[...] = jnp.concatenate([i1, i2], axis=1)

    w_full = jnp.where(lane == i1, w1, 0.0) + jnp.where(lane == i2, w2, 0.0)

    keys = lax.bitcast_convert_type(w_full, jnp.int32)
    lo = jnp.zeros((1, E), jnp.int32)
    for b in range(30, -1, -1):
        trial = lo | (1 << b)
        cnt = jnp.sum((keys >= trial).astype(jnp.int32), axis=0, keepdims=True)
        lo = jnp.where(cnt >= cap, trial, lo)
    tau = lo
    g = jnp.sum((keys > tau).astype(jnp.int32), axis=0, keepdims=True)
    tie = (keys == tau)
    tie_rank = _cumsum_excl(tie.astype(jnp.int32), T)
    keep = (keys > tau) | (tie & (tie_rank < (cap - g)))
    wk = jnp.where(keep, w_full, 0.0)
    wk_ref[...] = wk

    pos = (wk > 0).astype(jnp.int32)
    rank_ref[...] = _cumsum_excl(pos, T)
    hb_ref[...] = h.astype(jnp.bfloat16)


def _dense_body(cap, nf, hb_ref, wk_ref, rank_ref, wg_ref, wu_ref, wd_ref,
                out_ref, q_scr, xs_scr, ws_scr, ys_scr):
    e = pl.program_id(0)
    f = pl.program_id(1)
    T, D = hb_ref.shape
    E = wk_ref.shape[1]

    @pl.when(jnp.logical_and(e == 0, f == 0))
    def _init():
        out_ref[...] = jnp.zeros_like(out_ref)

    @pl.when(f == 0)
    def _dispatch():
        lane = lax.broadcasted_iota(jnp.int32, (T, E), 1)
        oh = (lane == e)
        w_col = jnp.sum(jnp.where(oh, wk_ref[...], 0.0), axis=1,
                        keepdims=True)
        rank_col = jnp.sum(jnp.where(oh, rank_ref[...], 0), axis=1,
                           keepdims=True)
        slot = lax.broadcasted_iota(jnp.int32, (T, cap), 1)
        q01 = jnp.logical_and(slot == rank_col, w_col > 0.0)
        q01f = q01.astype(jnp.float32)
        q_scr[...] = q01f.astype(jnp.bfloat16)
        ws_scr[...] = lax.dot_general(
            q01f, w_col, (((0,), (0,)), ((), ())),
            preferred_element_type=jnp.float32,
            precision=lax.Precision.HIGHEST)
        xs = lax.dot_general(
            q_scr[...], hb_ref[...], (((0,), (0,)), ((), ())),
            preferred_element_type=jnp.float32)
        xs_scr[...] = xs.astype(jnp.bfloat16)

    xs = xs_scr[...]
    wgb = wg_ref[0].astype(jnp.bfloat16)
    wub = wu_ref[0].astype(jnp.bfloat16)
    wdb = wd_ref[0].astype(jnp.bfloat16)
    g = jnp.dot(xs, wgb, preferred_element_type=jnp.float32)
    u = jnp.dot(xs, wub, preferred_element_type=jnp.float32)
    act = (g * (1.0 / (1.0 + jnp.exp(-g)))) * u
    contrib = jnp.dot(act.astype(jnp.bfloat16), wdb,
                      preferred_element_type=jnp.float32)

    @pl.when(f == 0)
    def _y0():
        ys_scr[...] = contrib

    @pl.when(f != 0)
    def _yacc():
        ys_scr[...] += contrib

    @pl.when(f == nf - 1)
    def _combine():
        ysw = (ys_scr[...] * ws_scr[...]).astype(jnp.bfloat16)
        out_ref[...] += jnp.dot(q_scr[...], ysw,
                                preferred_element_type=jnp.float32)


def kernel(hidden_states, w_gate, w_gate_proj, w_up_proj, w_down_proj):
    T, D = hidden_states.shape
    E = w_gate.shape[1]
    DFF = w_gate_proj.shape[2]
    CAP = int(math.ceil(T * _K / E * 1.25))
    NF = 2
    FBLK = DFF // NF

    ids, wk, rank, hb = pl.pallas_call(
        functools.partial(_routing_body, CAP),
        out_shape=(
            jax.ShapeDtypeStruct((T, _K), jnp.int32),
            jax.ShapeDtypeStruct((T, E), jnp.float32),
            jax.ShapeDtypeStruct((T, E), jnp.int32),
            jax.ShapeDtypeStruct((T, D), jnp.bfloat16),
        ),
    )(hidden_states, w_gate)

    out = pl.pallas_call(
        functools.partial(_dense_body, CAP, NF),
        grid=(E, NF),
        in_specs=[
            pl.BlockSpec((T, D), lambda e, f: (0, 0)),
            pl.BlockSpec((T, E), lambda e, f: (0, 0)),
            pl.BlockSpec((T, E), lambda e, f: (0, 0)),
            pl.BlockSpec((1, D, FBLK), lambda e, f: (e, 0, f)),
            pl.BlockSpec((1, D, FBLK), lambda e, f: (e, 0, f)),
            pl.BlockSpec((1, FBLK, D), lambda e, f: (e, f, 0)),
        ],
        out_specs=pl.BlockSpec((T, D), lambda e, f: (0, 0)),
        out_shape=jax.ShapeDtypeStruct((T, D), jnp.float32),
        scratch_shapes=[
            pltpu.VMEM((T, CAP), jnp.bfloat16),
            pltpu.VMEM((CAP, D), jnp.bfloat16),
            pltpu.VMEM((CAP, 1), jnp.float32),
            pltpu.VMEM((CAP, D), jnp.float32),
        ],
    )(hb, wk, rank, w_gate_proj, w_up_proj, w_down_proj)

    return out, ids
